# baseline (speedup 1.0000x reference)
import functools

import jax
import jax.numpy as jnp
from jax import lax
from jax.experimental import pallas as pl
from jax.experimental.pallas import tpu as pltpu
from jax.experimental.pallas import tpu_sc as plsc

N_USERS_K = 5000
N_ITEMS_K = 5000
N_TOT = N_USERS_K + N_ITEMS_K
D = 128
E_EDGES = 320000

NC = 2
NS = 16
NW = NC * NS
CHUNK = 128
CPW = 80
E_PAD = NW * CPW * CHUNK
ROWS_PER_TILE = 624


def _sc_round_body(emb_hbm, src_hbm, dst_hbm, w_hbm, out_hbm,
                   acc, src_sl, dst0, dst1, w0, w1, rows0, rows1,
                   gsem0, gsem1, ssem0, ssem1, dsem0, dsem1,
                   wsem0, wsem1):
    cid = lax.axis_index("c")
    sid = lax.axis_index("s")
    wid = sid * NC + cid
    rows = (rows0, rows1)
    dstb = (dst0, dst1)
    wb = (w0, w1)
    gsem = (gsem0, gsem1)
    ssem = (ssem0, ssem1)
    dsem = (dsem0, dsem1)
    wsem = (wsem0, wsem1)

    pltpu.sync_copy(src_hbm.at[wid], src_sl)

    def zero_rows(r, carry):
        for l in range(D // 16):
            rows0[r, pl.ds(l * 16, 16)] = jnp.zeros((16,), jnp.float32)
        return carry
    lax.fori_loop(0, CHUNK, zero_rows, 0)
    base_row = sid * ROWS_PER_TILE
    for k in range(-(-ROWS_PER_TILE // CHUNK)):
        nr = min(CHUNK, ROWS_PER_TILE - k * CHUNK)
        pltpu.sync_copy(rows0.at[pl.ds(0, nr)],
                        acc.at[pl.ds(base_row + k * CHUNK, nr)])
    tail_base = NS * ROWS_PER_TILE
    tail_rows = N_TOT - tail_base

    @pl.when(sid == 0)
    def _zero_tail():
        pltpu.sync_copy(rows0.at[pl.ds(0, tail_rows)],
                        acc.at[pl.ds(tail_base, tail_rows)])
    plsc.subcore_barrier()

    def start_gather(j, b):
        pltpu.async_copy(emb_hbm.at[src_sl.at[j]], rows[b], gsem[b])

    def wait_gather(b):
        pltpu.make_async_copy(emb_hbm.at[src_sl.at[0]], rows[b],
                              gsem[b]).wait()

    def start_dst(j, b):
        pltpu.async_copy(dst_hbm.at[wid, j], dstb[b].at[0], dsem[b])

    def wait_dst(b):
        pltpu.make_async_copy(dst_hbm.at[0, 0], dstb[b].at[0],
                              dsem[b]).wait()

    def start_w(j, b):
        pltpu.async_copy(w_hbm.at[wid, j], wb[b].at[0], wsem[b])

    def wait_w(b):
        pltpu.make_async_copy(w_hbm.at[0, 0], wb[b].at[0], wsem[b]).wait()

    def start_scatter(b):
        pltpu.async_copy(rows[b], acc.at[dstb[b].at[0]], ssem[b], add=True)

    def wait_scatter(b):
        pltpu.make_async_copy(rows[b], acc.at[dstb[b].at[0]],
                              ssem[b]).wait()

    def scale(j, b):
        rv = rows[b]

        wrow = wb[b]

        def scale_group(g, c2):
            wvec = wrow[0, pl.ds(g * 16, 16)]
            for e in range(16):
                we = wvec[e]
                row = g * 16 + e
                for l in range(D // 16):
                    rv[row, pl.ds(l * 16, 16)] = (
                        rv[row, pl.ds(l * 16, 16)] * we)
            return c2
        lax.fori_loop(0, CHUNK // 16, scale_group, 0)

    start_dst(0, 0)
    start_w(0, 0)
    start_gather(0, 0)
    wait_gather(0)
    start_dst(1, 1)
    start_w(1, 1)
    start_gather(1, 1)
    wait_w(0)
    scale(0, 0)
    wait_dst(0)
    start_scatter(0)

    def step(j, b):
        wait_gather(b)
        wait_scatter(1 - b)
        start_dst(j + 1, 1 - b)
        start_w(j + 1, 1 - b)
        start_gather(j + 1, 1 - b)
        wait_w(b)
        scale(j, b)
        wait_dst(b)
        start_scatter(b)

    def body(t, carry):
        step(2 * t + 1, 1)
        step(2 * t + 2, 0)
        return carry
    lax.fori_loop(0, (CPW - 2) // 2, body, 0)

    wait_gather(1)
    wait_w(1)
    scale(CPW - 1, 1)
    wait_dst(1)
    start_scatter(1)
    wait_scatter(0)
    wait_scatter(1)
    plsc.subcore_barrier()

    pltpu.sync_copy(acc.at[pl.ds(base_row, ROWS_PER_TILE)],
                    out_hbm.at[pl.ds(cid * N_TOT + base_row, ROWS_PER_TILE)])

    @pl.when(sid == 0)
    def _write_tail():
        pltpu.sync_copy(acc.at[pl.ds(tail_base, tail_rows)],
                        out_hbm.at[pl.ds(cid * N_TOT + tail_base, tail_rows)])


@jax.jit
def _sc_round(emb, src3d, dst3d, w3d):
    mesh = plsc.VectorSubcoreMesh(core_axis_name="c", subcore_axis_name="s")
    return pl.kernel(
        _sc_round_body,
        out_type=jax.ShapeDtypeStruct((NC * N_TOT, D), jnp.float32),
        mesh=mesh,
        scratch_types=[
            pltpu.VMEM_SHARED((N_TOT, D), jnp.float32),
            pltpu.VMEM((CPW, CHUNK), jnp.int32),
            pltpu.VMEM((1, CHUNK), jnp.int32),
            pltpu.VMEM((1, CHUNK), jnp.int32),
            pltpu.VMEM((1, CHUNK), jnp.float32),
            pltpu.VMEM((1, CHUNK), jnp.float32),
            pltpu.VMEM((CHUNK, D), jnp.float32),
            pltpu.VMEM((CHUNK, D), jnp.float32),
            pltpu.SemaphoreType.DMA,
            pltpu.SemaphoreType.DMA,
            pltpu.SemaphoreType.DMA,
            pltpu.SemaphoreType.DMA,
            pltpu.SemaphoreType.DMA,
            pltpu.SemaphoreType.DMA,
            pltpu.SemaphoreType.DMA,
            pltpu.SemaphoreType.DMA,
        ],
    )(emb, src3d, dst3d, w3d)


def _add2_body(a_ref, b_ref, o_ref):
    o_ref[...] = a_ref[...] + b_ref[...]


def _final_body(e0_ref, e1_ref, p0_ref, p1_ref, o_ref):
    o_ref[...] = (e0_ref[...] + e1_ref[...] + p0_ref[...] + p1_ref[...]) * (1.0 / 3.0)


_TC_BLK = 1000


def _tc_specs(n_in):
    spec = pl.BlockSpec((_TC_BLK, D), lambda i: (i, 0))
    return dict(
        grid=(N_TOT // _TC_BLK,),
        in_specs=[spec] * n_in,
        out_specs=spec,
        out_shape=jax.ShapeDtypeStruct((N_TOT, D), jnp.float32),
    )


@jax.jit
def _combine2(p):
    return pl.pallas_call(_add2_body, **_tc_specs(2))(p[:N_TOT], p[N_TOT:])


@jax.jit
def _final(emb0, emb1, p2):
    return pl.pallas_call(_final_body, **_tc_specs(4))(
        emb0, emb1, p2[:N_TOT], p2[N_TOT:])


def kernel(edge_index, edge_weight, user_emb, item_emb):
    emb0 = jnp.concatenate([user_emb, item_emb], axis=0)
    dst = edge_index[0]
    src = edge_index[1]
    pad = E_PAD - E_EDGES
    src3d = jnp.pad(src, (0, pad)).reshape(NW, CPW, CHUNK)
    dst3d = jnp.pad(dst, (0, pad)).reshape(NW, CPW, CHUNK)
    w3d = jnp.pad(edge_weight, (0, pad)).reshape(NW, CPW, CHUNK)

    p1 = _sc_round(emb0, src3d, dst3d, w3d)
    emb1 = _combine2(p1)
    p2 = _sc_round(emb1, src3d, dst3d, w3d)
    out = _final(emb0, emb1, p2)
    return (out[:N_USERS_K], out[N_USERS_K:])


# speedup vs baseline: 1.1335x; 1.1335x over previous
import functools

import jax
import jax.numpy as jnp
from jax import lax
from jax.experimental import pallas as pl
from jax.experimental.pallas import tpu as pltpu
from jax.experimental.pallas import tpu_sc as plsc

N_USERS_K = 5000
N_ITEMS_K = 5000
N_TOT = N_USERS_K + N_ITEMS_K
D = 128
E_EDGES = 320000

NC = 2
NS = 16
NW = NC * NS
CHUNK = 128
CPW0 = 128
CPW1 = 32
NCH = NS * (CPW0 + CPW1)
E_PAD = NCH * CHUNK
ROWS_PER_TILE = 624


def _sc_round_body(emb_hbm, src_hbm, dst_hbm, w_hbm, out_hbm,
                   acc, src_sl, dst0, dst1, w0, w1, rows0, rows1,
                   gsem0, gsem1, ssem0, ssem1, dsem0, dsem1,
                   wsem0, wsem1):
    cid = lax.axis_index("c")
    sid = lax.axis_index("s")
    wid = sid * NC + cid
    rows = (rows0, rows1)
    dstb = (dst0, dst1)
    wb = (w0, w1)
    gsem = (gsem0, gsem1)
    ssem = (ssem0, ssem1)
    dsem = (dsem0, dsem1)
    wsem = (wsem0, wsem1)


    def zero_rows(r, carry):
        for l in range(D // 16):
            rows0[r, pl.ds(l * 16, 16)] = jnp.zeros((16,), jnp.float32)
        return carry
    lax.fori_loop(0, CHUNK, zero_rows, 0)
    base_row = sid * ROWS_PER_TILE
    for k in range(-(-ROWS_PER_TILE // CHUNK)):
        nr = min(CHUNK, ROWS_PER_TILE - k * CHUNK)
        pltpu.sync_copy(rows0.at[pl.ds(0, nr)],
                        acc.at[pl.ds(base_row + k * CHUNK, nr)])
    tail_base = NS * ROWS_PER_TILE
    tail_rows = N_TOT - tail_base

    @pl.when(sid == 0)
    def _zero_tail():
        pltpu.sync_copy(rows0.at[pl.ds(0, tail_rows)],
                        acc.at[pl.ds(tail_base, tail_rows)])
    plsc.subcore_barrier()

    def start_gather(j, b):
        pltpu.async_copy(emb_hbm.at[src_sl.at[j]], rows[b], gsem[b])

    def wait_gather(b):
        pltpu.make_async_copy(emb_hbm.at[src_sl.at[0]], rows[b],
                              gsem[b]).wait()

    def start_dst(r, b):
        pltpu.async_copy(dst_hbm.at[r], dstb[b].at[0], dsem[b])

    def wait_dst(b):
        pltpu.make_async_copy(dst_hbm.at[0], dstb[b].at[0],
                              dsem[b]).wait()

    def start_w(r, b):
        pltpu.async_copy(w_hbm.at[r], wb[b].at[0], wsem[b])

    def wait_w(b):
        pltpu.make_async_copy(w_hbm.at[0], wb[b].at[0], wsem[b]).wait()

    def start_scatter(b):
        pltpu.async_copy(rows[b], acc.at[dstb[b].at[0]], ssem[b], add=True)

    def wait_scatter(b):
        pltpu.make_async_copy(rows[b], acc.at[dstb[b].at[0]],
                              ssem[b]).wait()

    def scale(b):
        rv = rows[b]
        wrow = wb[b]

        def scale_group(g, c2):
            wvec = wrow[0, pl.ds(g * 16, 16)]
            for e in range(16):
                we = wvec[e]
                row = g * 16 + e
                for l in range(D // 16):
                    rv[row, pl.ds(l * 16, 16)] = (
                        rv[row, pl.ds(l * 16, 16)] * we)
            return c2
        lax.fori_loop(0, CHUNK // 16, scale_group, 0)

    def run_core(cpw, start):
        pltpu.sync_copy(src_hbm.at[pl.ds(start, cpw)],
                        src_sl.at[pl.ds(0, cpw)])

        start_dst(start + 0, 0)
        start_w(start + 0, 0)
        start_gather(0, 0)
        wait_gather(0)
        start_dst(start + 1, 1)
        start_w(start + 1, 1)
        start_gather(1, 1)
        wait_w(0)
        scale(0)
        wait_dst(0)
        start_scatter(0)

        def step(j, b):
            wait_gather(b)
            wait_scatter(1 - b)
            start_dst(start + j + 1, 1 - b)
            start_w(start + j + 1, 1 - b)
            start_gather(j + 1, 1 - b)
            wait_w(b)
            scale(b)
            wait_dst(b)
            start_scatter(b)

        def body(t, carry):
            step(2 * t + 1, 1)
            step(2 * t + 2, 0)
            return carry
        lax.fori_loop(0, (cpw - 2) // 2, body, 0)

        wait_gather(1)
        wait_w(1)
        scale(1)
        wait_dst(1)
        start_scatter(1)
        wait_scatter(0)
        wait_scatter(1)

    @pl.when(cid == 0)
    def _core0():
        run_core(CPW0, sid * CPW0)

    @pl.when(cid == 1)
    def _core1():
        run_core(CPW1, NS * CPW0 + sid * CPW1)
    plsc.subcore_barrier()

    pltpu.sync_copy(acc.at[pl.ds(base_row, ROWS_PER_TILE)],
                    out_hbm.at[pl.ds(cid * N_TOT + base_row, ROWS_PER_TILE)])

    @pl.when(sid == 0)
    def _write_tail():
        pltpu.sync_copy(acc.at[pl.ds(tail_base, tail_rows)],
                        out_hbm.at[pl.ds(cid * N_TOT + tail_base, tail_rows)])


@jax.jit
def _sc_round(emb, src3d, dst3d, w3d):
    mesh = plsc.VectorSubcoreMesh(core_axis_name="c", subcore_axis_name="s")
    return pl.kernel(
        _sc_round_body,
        out_type=jax.ShapeDtypeStruct((NC * N_TOT, D), jnp.float32),
        mesh=mesh,
        scratch_types=[
            pltpu.VMEM_SHARED((N_TOT, D), jnp.float32),
            pltpu.VMEM((CPW0, CHUNK), jnp.int32),
            pltpu.VMEM((1, CHUNK), jnp.int32),
            pltpu.VMEM((1, CHUNK), jnp.int32),
            pltpu.VMEM((1, CHUNK), jnp.float32),
            pltpu.VMEM((1, CHUNK), jnp.float32),
            pltpu.VMEM((CHUNK, D), jnp.float32),
            pltpu.VMEM((CHUNK, D), jnp.float32),
            pltpu.SemaphoreType.DMA,
            pltpu.SemaphoreType.DMA,
            pltpu.SemaphoreType.DMA,
            pltpu.SemaphoreType.DMA,
            pltpu.SemaphoreType.DMA,
            pltpu.SemaphoreType.DMA,
            pltpu.SemaphoreType.DMA,
            pltpu.SemaphoreType.DMA,
        ],
    )(emb, src3d, dst3d, w3d)


def _add2_body(a_ref, b_ref, o_ref):
    o_ref[...] = a_ref[...] + b_ref[...]


def _final_body(e0_ref, e1_ref, p0_ref, p1_ref, o_ref):
    o_ref[...] = (e0_ref[...] + e1_ref[...] + p0_ref[...] + p1_ref[...]) * (1.0 / 3.0)


_TC_BLK = 1000


def _tc_specs(n_in):
    spec = pl.BlockSpec((_TC_BLK, D), lambda i: (i, 0))
    return dict(
        grid=(N_TOT // _TC_BLK,),
        in_specs=[spec] * n_in,
        out_specs=spec,
        out_shape=jax.ShapeDtypeStruct((N_TOT, D), jnp.float32),
    )


@jax.jit
def _combine2(p):
    return pl.pallas_call(_add2_body, **_tc_specs(2))(p[:N_TOT], p[N_TOT:])


@jax.jit
def _final(emb0, emb1, p2):
    return pl.pallas_call(_final_body, **_tc_specs(4))(
        emb0, emb1, p2[:N_TOT], p2[N_TOT:])


def kernel(edge_index, edge_weight, user_emb, item_emb):
    emb0 = jnp.concatenate([user_emb, item_emb], axis=0)
    dst = edge_index[0]
    src = edge_index[1]
    pad = E_PAD - E_EDGES
    src3d = jnp.pad(src, (0, pad)).reshape(NCH, CHUNK)
    dst3d = jnp.pad(dst, (0, pad)).reshape(NCH, CHUNK)
    w3d = jnp.pad(edge_weight, (0, pad)).reshape(NCH, CHUNK)

    p1 = _sc_round(emb0, src3d, dst3d, w3d)
    emb1 = _combine2(p1)
    p2 = _sc_round(emb1, src3d, dst3d, w3d)
    out = _final(emb0, emb1, p2)
    return (out[:N_USERS_K], out[N_USERS_K:])
